# Initial kernel scaffold; baseline (speedup 1.0000x reference)
import jax
import jax.numpy as jnp
from jax.experimental import pallas as pl


def kernel(features, edge_index, W1, b1, W2, b2):
    raise NotImplementedError("write your pallas kernel here")


# speedup vs baseline: 11.6035x; 11.6035x over previous
import functools

import jax
import jax.numpy as jnp
from jax import lax
from jax.experimental import pallas as pl
from jax.experimental.pallas import tpu as pltpu
from jax.experimental.pallas import tpu_sc as plsc

N = 10000
E = 320000
D_IN = 128
D_HID = 256
D_OUT = 128

NC = 2
NS = 16
NW = NC * NS
EW = E // NW
CH = 125
NCHUNK = EW // CH
STRIPE = 624
ZROWS = 640

_MESH = plsc.VectorSubcoreMesh(core_axis_name="c", subcore_axis_name="s")


@functools.partial(
    pl.kernel,
    out_type=jax.ShapeDtypeStruct((NC * N, 128), jnp.float32),
    mesh=_MESH,
    scratch_types=(
        pltpu.VMEM((NCHUNK, CH), jnp.int32),
        pltpu.VMEM((CH, 128), jnp.float32),
        pltpu.VMEM_SHARED((N, 128), jnp.float32),
    ),
)
def _sc_degrees(idx_hbm, ones_hbm, zeros_hbm, deg_out, idx_v, ones_v, acc):
    cid = lax.axis_index("c")
    sid = lax.axis_index("s")
    wid = sid * NC + cid

    pltpu.sync_copy(zeros_hbm, acc.at[pl.ds(sid * STRIPE, ZROWS)])
    pltpu.sync_copy(idx_hbm.at[pl.ds(wid * NCHUNK, NCHUNK)], idx_v)
    pltpu.sync_copy(ones_hbm, ones_v)
    plsc.subcore_barrier()

    def body(j, _):
        pltpu.sync_copy(ones_v, acc.at[idx_v.at[j]], add=True)
        return _

    lax.fori_loop(0, NCHUNK, body, None)
    plsc.subcore_barrier()

    pltpu.sync_copy(acc.at[pl.ds(sid * STRIPE, ZROWS)],
                    deg_out.at[pl.ds(cid * N + sid * STRIPE, ZROWS)])


@functools.partial(
    pl.kernel,
    out_type=jax.ShapeDtypeStruct((NC * N, 128), jnp.float32),
    mesh=_MESH,
    scratch_types=(
        pltpu.VMEM((NCHUNK, CH), jnp.int32),
        pltpu.VMEM((NCHUNK, CH), jnp.int32),
        pltpu.VMEM((CH, 128), jnp.float32),
        pltpu.VMEM_SHARED((N, 128), jnp.float32),
        pltpu.SemaphoreType.DMA,
    ),
)
def _sc_spmm(h_hbm, src_hbm, dst_hbm, zeros_hbm, out_hbm,
             src_v, dst_v, rows_v, acc, sem):
    cid = lax.axis_index("c")
    sid = lax.axis_index("s")
    wid = sid * NC + cid

    pltpu.sync_copy(zeros_hbm, acc.at[pl.ds(sid * STRIPE, ZROWS)])
    pltpu.sync_copy(src_hbm.at[pl.ds(wid * NCHUNK, NCHUNK)], src_v)
    pltpu.sync_copy(dst_hbm.at[pl.ds(wid * NCHUNK, NCHUNK)], dst_v)
    plsc.subcore_barrier()

    def body(j, _):
        pltpu.async_copy(h_hbm.at[src_v.at[j]], rows_v, sem).wait()
        pltpu.sync_copy(rows_v, acc.at[dst_v.at[j]], add=True)
        return _

    lax.fori_loop(0, NCHUNK, body, None)
    plsc.subcore_barrier()

    pltpu.sync_copy(acc.at[pl.ds(sid * STRIPE, ZROWS)],
                    out_hbm.at[pl.ds(cid * N + sid * STRIPE, ZROWS)])


_BLK = 1000


def _norms_body(ds0, ds1, dd0, dd1, feat, xs_o, ns_o, nd_o):
    ns = lax.rsqrt(jnp.maximum(ds0[...] + ds1[...], 1.0))
    nd = lax.rsqrt(jnp.maximum(dd0[...] + dd1[...], 1.0))
    ns_o[...] = ns
    nd_o[...] = nd
    xs_o[...] = feat[...] * ns[:, 0:1]


_NORMW = 128


def _mid_body(p0, p1, nd, ns, w1, b1, w2, o):
    agg = (p0[...] + p1[...]) * nd[:, 0:1]
    x1 = jnp.maximum(
        jnp.dot(agg, w1[...], preferred_element_type=jnp.float32)
        + b1[...][None, :], 0.0)
    o[...] = jnp.dot(x1, w2[...], preferred_element_type=jnp.float32) * ns[:, 0:1]


def _final_body(p0, p1, nd, b2, o):
    agg = (p0[...] + p1[...]) * nd[:, 0:1]
    o[...] = jnp.maximum(agg + b2[...][None, :], 0.0)


def _row_spec(width):
    return pl.BlockSpec((_BLK, width), lambda i: (i, 0))


def _full_spec(shape):
    nd = len(shape)
    return pl.BlockSpec(shape, lambda i: (0,) * nd)


_tc_norms = pl.pallas_call(
    _norms_body,
    grid=(N // _BLK,),
    in_specs=[_row_spec(_NORMW), _row_spec(_NORMW), _row_spec(_NORMW),
              _row_spec(_NORMW), _row_spec(128)],
    out_specs=[_row_spec(128), _row_spec(_NORMW), _row_spec(_NORMW)],
    out_shape=(jax.ShapeDtypeStruct((N, 128), jnp.float32),
               jax.ShapeDtypeStruct((N, _NORMW), jnp.float32),
               jax.ShapeDtypeStruct((N, _NORMW), jnp.float32)),
)

_tc_mid = pl.pallas_call(
    _mid_body,
    grid=(N // _BLK,),
    in_specs=[_row_spec(128), _row_spec(128), _row_spec(_NORMW),
              _row_spec(_NORMW), _full_spec((D_IN, D_HID)),
              _full_spec((D_HID,)), _full_spec((D_HID, D_OUT))],
    out_specs=_row_spec(128),
    out_shape=jax.ShapeDtypeStruct((N, 128), jnp.float32),
)

_tc_final = pl.pallas_call(
    _final_body,
    grid=(N // _BLK,),
    in_specs=[_row_spec(128), _row_spec(128), _row_spec(_NORMW),
              _full_spec((D_OUT,))],
    out_specs=_row_spec(128),
    out_shape=jax.ShapeDtypeStruct((N, 128), jnp.float32),
)


def kernel(features, edge_index, W1, b1, W2, b2):
    src = edge_index[0].astype(jnp.int32).reshape(NW * NCHUNK, CH)
    dst = edge_index[1].astype(jnp.int32).reshape(NW * NCHUNK, CH)

    ones128 = jnp.ones((CH, 128), jnp.float32)
    zeros128 = jnp.zeros((ZROWS, 128), jnp.float32)

    degs = _sc_degrees(src, ones128, zeros128)
    degd = _sc_degrees(dst, ones128, zeros128)
    xs, ns16, nd16 = _tc_norms(degs[:N], degs[N:], degd[:N], degd[N:],
                               features)

    p1 = _sc_spmm(xs, src, dst, zeros128)
    h2s = _tc_mid(p1[:N], p1[N:], nd16, ns16, W1, b1, W2)

    p2 = _sc_spmm(h2s, src, dst, zeros128)
    return _tc_final(p2[:N], p2[N:], nd16, b2)
